# Initial kernel scaffold; baseline (speedup 1.0000x reference)
import jax
import jax.numpy as jnp
from jax.experimental import pallas as pl


def kernel(x, moe_router, moe_w13, moe_w2):
    raise NotImplementedError("write your pallas kernel here")


# speedup vs baseline: 3.5587x; 3.5587x over previous
import functools

import jax
import jax.numpy as jnp
from jax.experimental import pallas as pl
from jax.experimental.pallas import tpu as pltpu

E = 8
TOPK = 2
TILE = 256
D = 1024
F = 4096
MOE_D = 2048


def _gmm_body(eot_ref, xd_ref, w13_ref, w2_ref, wrow_ref, y_ref):
    h = jnp.dot(xd_ref[...], w13_ref[0], preferred_element_type=jnp.float32)
    gate = h[:, :MOE_D]
    up = h[:, MOE_D:]
    a = jax.nn.silu(gate) * up
    y = jnp.dot(a, w2_ref[0], preferred_element_type=jnp.float32)
    y_ref[...] = y * wrow_ref[...]


def _grouped_mlp(eot, xd, w13, w2, wrow, nt):
    grid_spec = pltpu.PrefetchScalarGridSpec(
        num_scalar_prefetch=1,
        grid=(nt,),
        in_specs=[
            pl.BlockSpec((TILE, D), lambda i, eot: (i, 0)),
            pl.BlockSpec((1, D, F), lambda i, eot: (eot[i], 0, 0)),
            pl.BlockSpec((1, MOE_D, D), lambda i, eot: (eot[i], 0, 0)),
            pl.BlockSpec((TILE, 1), lambda i, eot: (i, 0)),
        ],
        out_specs=pl.BlockSpec((TILE, D), lambda i, eot: (i, 0)),
    )
    return pl.pallas_call(
        _gmm_body,
        grid_spec=grid_spec,
        out_shape=jax.ShapeDtypeStruct((nt * TILE, D), jnp.float32),
        compiler_params=pltpu.CompilerParams(
            vmem_limit_bytes=110 * 1024 * 1024,
        ),
    )(eot, xd, w13, w2, wrow)


def kernel(x, moe_router, moe_w13, moe_w2):
    b, s, d = x.shape
    tokens = b * s
    x_flat = x.reshape(tokens, d)
    nt = tokens * TOPK // TILE + E - 1
    r_pad = nt * TILE

    logits = x_flat @ moe_router
    topk_logits, topk_idx = jax.lax.top_k(logits, TOPK)
    topk_w = jax.nn.softmax(topk_logits, axis=-1)
    e_slot = jnp.concatenate([topk_idx[:, 0], topk_idx[:, 1]])
    w_slot = jnp.concatenate([topk_w[:, 0], topk_w[:, 1]])
    oh = (e_slot[:, None] == jnp.arange(E)[None, :]).astype(jnp.int32)
    pref = jnp.cumsum(oh, axis=0)
    cnt = pref[-1]
    padded = ((cnt + TILE - 1) // TILE) * TILE
    pad_off = jnp.concatenate([jnp.zeros((1,), jnp.int32),
                               jnp.cumsum(padded)[:-1].astype(jnp.int32)])
    rank = jnp.sum(pref * oh, axis=1) - 1
    pos = pad_off[e_slot] + rank
    src = jnp.zeros((r_pad,), jnp.int32).at[pos].set(
        jnp.arange(tokens * TOPK, dtype=jnp.int32) % tokens)
    wdisp = jnp.zeros((r_pad,), jnp.float32).at[pos].set(w_slot)
    pad_end = (pad_off + padded).astype(jnp.int32)
    tile_start = jnp.arange(nt, dtype=jnp.int32) * TILE
    eot = jnp.minimum(
        jnp.sum((pad_end[None, :] <= tile_start[:, None]).astype(jnp.int32),
                axis=1), E - 1).astype(jnp.int32)

    xd = x_flat[src]

    y = _grouped_mlp(eot, xd, moe_w13, moe_w2, wdisp[:, None], nt)

    out_flat = y[pos[:tokens]] + y[pos[tokens:]]
    return out_flat.reshape(b, s, d)
